# baseline (speedup 1.0000x reference)
import functools

import jax
import jax.numpy as jnp
from jax import lax
from jax.experimental import pallas as pl
from jax.experimental.pallas import tpu as pltpu
from jax.experimental.pallas import tpu_sc as plsc

VOCAB = 100000
HIDDEN = 128
MAX_POS = 512
EPS = 1e-12

NC, NS, L = 2, 16, 16
NW = NC * NS
N_TOK = 64 * 512
TPW = N_TOK // NW
C = 128
NCHUNK = TPW // C
NSL = HIDDEN // L
NG = C // L
W = 17


def _tree8(v):
    return ((v[0] + v[1]) + (v[2] + v[3])) + ((v[4] + v[5]) + (v[6] + v[7]))


def _tec_body(ids_hbm, tt_hbm, word_hbm, pos_hbm, type_hbm, gamma_hbm,
              beta_hbm, out_hbm, pos_v, rows0, rows1, idx_all, tt_all,
              type_v, g_v, b_v, sbuf, qbuf, gsem0, gsem1, wsem0, wsem1):
    wid = lax.axis_index("s") * NC + lax.axis_index("c")
    base = wid * TPW

    pltpu.sync_copy(pos_hbm, pos_v)
    pltpu.sync_copy(type_hbm, type_v)
    pltpu.sync_copy(gamma_hbm, g_v)
    pltpu.sync_copy(beta_hbm, b_v)
    pltpu.sync_copy(ids_hbm.at[pl.ds(base, TPW)], idx_all)
    pltpu.sync_copy(tt_hbm.at[pl.ds(base, TPW)], tt_all)

    g = [g_v[pl.ds(L * j, L)] for j in range(NSL)]
    b = [b_v[pl.ds(L * j, L)] for j in range(NSL)]
    t0 = [type_v[0, pl.ds(L * j, L)] for j in range(NSL)]
    t1 = [type_v[1, pl.ds(L * j, L)] for j in range(NSL)]
    ci = lax.iota(jnp.int32, L)
    ciw = ci * W

    def prep(c, rowsv, gsem):
        pltpu.async_copy(
            word_hbm.at[idx_all.at[pl.ds(c * C, C)]], rowsv, gsem)

    def gwait(c, rowsv, gsem):
        pltpu.make_async_copy(
            word_hbm.at[idx_all.at[pl.ds(c * C, C)]], rowsv, gsem).wait()

    def wb_start(c, rowsv, wsem):
        start = base + c * C
        pltpu.async_copy(rowsv, out_hbm.at[pl.ds(start, C)], wsem)

    def wb_wait(rowsv, wsem):
        pltpu.make_async_copy(rowsv, out_hbm.at[pl.ds(base, C)], wsem).wait()

    def compute(rowsv, c):
        prow_base = lax.rem(c, MAX_POS // C) * C
        tbase = c * C

        @plsc.parallel_loop(0, NG, 1, unroll=1)
        def grp(gi):
            gbase = gi * L
            sb = gi * (L * W)
            tg = tt_all[pl.ds(tbase + gbase, L)]
            for k in range(L):
                i = gbase + k
                p = prow_base + i
                is1 = tg[k] == 1
                xs = []
                for j in range(NSL):
                    sl = pl.ds(L * j, L)
                    tv = jnp.where(is1, t1[j], t0[j])
                    x = rowsv[i, sl] + pos_v[p, sl] + tv
                    rowsv[i, sl] = x
                    xs.append(x)
                s = _tree8(xs)
                q = _tree8([x * x for x in xs])
                plsc.store_scatter(sbuf, [ciw + (sb + k)], s)
                plsc.store_scatter(qbuf, [ciw + (sb + k)], q)
            vs = [plsc.load_gather(sbuf, [ci + (sb + W * l)])
                  for l in range(L)]
            vq = [plsc.load_gather(qbuf, [ci + (sb + W * l)])
                  for l in range(L)]
            tot = _tree8(vs[:8]) + _tree8(vs[8:])
            totq = _tree8(vq[:8]) + _tree8(vq[8:])
            mu = tot * (1.0 / HIDDEN)
            var = totq * (1.0 / HIDDEN) - mu * mu
            v = var + EPS
            iy = jnp.int32(0x5F3759DF) - lax.shift_right_arithmetic(
                plsc.bitcast(v, jnp.int32), 1)
            y = plsc.bitcast(iy, jnp.float32)
            h = 0.5 * v
            y = y * (1.5 - h * y * y)
            y = y * (1.5 - h * y * y)
            y = y * (1.5 - h * y * y)
            nbv = -mu * y
            for k in range(L):
                i = gbase + k
                yk = y[k]
                nk = nbv[k]
                for j in range(NSL):
                    sl = pl.ds(L * j, L)
                    x = rowsv[i, sl]
                    rowsv[i, sl] = (x * yk + nk) * g[j] + b[j]

    prep(0, rows0, gsem0)

    def pair(h, carry):
        c0 = 2 * h

        @pl.when(h > 0)
        def _():
            wb_wait(rows1, wsem1)

        prep(c0 + 1, rows1, gsem1)
        gwait(c0, rows0, gsem0)
        compute(rows0, c0)
        wb_start(c0, rows0, wsem0)

        @pl.when(h < NCHUNK // 2 - 1)
        def _():
            wb_wait(rows0, wsem0)
            prep(c0 + 2, rows0, gsem0)

        gwait(c0 + 1, rows1, gsem1)
        compute(rows1, c0 + 1)
        wb_start(c0 + 1, rows1, wsem1)
        return carry

    lax.fori_loop(0, NCHUNK // 2, pair, 0)
    wb_wait(rows0, wsem0)
    wb_wait(rows1, wsem1)


@jax.jit
def _bert_embed_sc(ids_flat, tt_flat, word_table, pos_table, type_table,
                   gamma, beta):
    mesh = plsc.VectorSubcoreMesh(core_axis_name="c", subcore_axis_name="s")
    run = functools.partial(
        pl.kernel,
        out_type=jax.ShapeDtypeStruct((N_TOK, HIDDEN), jnp.float32),
        mesh=mesh,
        compiler_params=pltpu.CompilerParams(needs_layout_passes=False),
        scratch_types=[
            pltpu.VMEM((MAX_POS, HIDDEN), jnp.float32),
            pltpu.VMEM((C, HIDDEN), jnp.float32),
            pltpu.VMEM((C, HIDDEN), jnp.float32),
            pltpu.VMEM((TPW,), jnp.int32),
            pltpu.VMEM((TPW,), jnp.int32),
            pltpu.VMEM((2, HIDDEN), jnp.float32),
            pltpu.VMEM((HIDDEN,), jnp.float32),
            pltpu.VMEM((HIDDEN,), jnp.float32),
            pltpu.VMEM((NG * L * W,), jnp.float32),
            pltpu.VMEM((NG * L * W,), jnp.float32),
            pltpu.SemaphoreType.DMA,
            pltpu.SemaphoreType.DMA,
            pltpu.SemaphoreType.DMA,
            pltpu.SemaphoreType.DMA,
        ],
    )(_tec_body)
    return run(ids_flat, tt_flat, word_table, pos_table, type_table,
               gamma, beta)


def kernel(input_ids, token_type_ids, word_table, pos_table, type_table,
           gamma, beta):
    B, S = input_ids.shape
    out = _bert_embed_sc(
        input_ids.reshape(-1).astype(jnp.int32),
        token_type_ids.reshape(-1).astype(jnp.int32),
        word_table, pos_table, type_table, gamma, beta)
    return out.reshape(B, S, HIDDEN)


# speedup vs baseline: 1.3959x; 1.3959x over previous
import functools

import jax
import jax.numpy as jnp
from jax import lax
from jax.experimental import pallas as pl
from jax.experimental.pallas import tpu as pltpu
from jax.experimental.pallas import tpu_sc as plsc

VOCAB = 100000
HIDDEN = 128
MAX_POS = 512
EPS = 1e-12

NC, NS, L = 2, 16, 16
NW = NC * NS
N_TOK = 64 * 512
TPW = N_TOK // NW
C = 128
NCHUNK = TPW // C


def _gather_body(ids_hbm, word_hbm, out_hbm, rows0, rows1, idx_all,
                 gsem0, gsem1, wsem0, wsem1):
    wid = lax.axis_index("s") * NC + lax.axis_index("c")
    base = wid * TPW
    pltpu.sync_copy(ids_hbm.at[pl.ds(base, TPW)], idx_all)

    def prep(c, rowsv, gsem):
        pltpu.async_copy(
            word_hbm.at[idx_all.at[pl.ds(c * C, C)]], rowsv, gsem)

    def gwait(c, rowsv, gsem):
        pltpu.make_async_copy(
            word_hbm.at[idx_all.at[pl.ds(c * C, C)]], rowsv, gsem).wait()

    def wb_start(c, rowsv, wsem):
        start = base + c * C
        pltpu.async_copy(rowsv, out_hbm.at[pl.ds(start, C)], wsem)

    def wb_wait(rowsv, wsem):
        pltpu.make_async_copy(rowsv, out_hbm.at[pl.ds(base, C)], wsem).wait()

    prep(0, rows0, gsem0)

    def pair(h, carry):
        c0 = 2 * h

        @pl.when(h > 0)
        def _():
            wb_wait(rows1, wsem1)

        prep(c0 + 1, rows1, gsem1)
        gwait(c0, rows0, gsem0)
        wb_start(c0, rows0, wsem0)

        @pl.when(h < NCHUNK // 2 - 1)
        def _():
            wb_wait(rows0, wsem0)
            prep(c0 + 2, rows0, gsem0)

        gwait(c0 + 1, rows1, gsem1)
        wb_start(c0 + 1, rows1, wsem1)
        return carry

    lax.fori_loop(0, NCHUNK // 2, pair, 0)
    wb_wait(rows0, wsem0)
    wb_wait(rows1, wsem1)


def _sc_gather(ids_flat, word_table):
    mesh = plsc.VectorSubcoreMesh(core_axis_name="c", subcore_axis_name="s")
    run = functools.partial(
        pl.kernel,
        out_type=jax.ShapeDtypeStruct((N_TOK, HIDDEN), jnp.float32),
        mesh=mesh,
        compiler_params=pltpu.CompilerParams(needs_layout_passes=False),
        scratch_types=[
            pltpu.VMEM((C, HIDDEN), jnp.float32),
            pltpu.VMEM((C, HIDDEN), jnp.float32),
            pltpu.VMEM((TPW,), jnp.int32),
            pltpu.SemaphoreType.DMA,
            pltpu.SemaphoreType.DMA,
            pltpu.SemaphoreType.DMA,
            pltpu.SemaphoreType.DMA,
        ],
    )(_gather_body)
    return run(ids_flat, word_table)


def _ln_body(x_ref, ttf_ref, pos_ref, type_ref, g_ref, b_ref, o_ref):
    x = x_ref[...]
    ttf = ttf_ref[...]
    t0 = type_ref[0, :][None, :]
    t1 = type_ref[1, :][None, :]
    x = x + pos_ref[...] + t0 + ttf * (t1 - t0)
    mu = jnp.mean(x, axis=-1, keepdims=True)
    xc = x - mu
    var = jnp.mean(xc * xc, axis=-1, keepdims=True)
    o_ref[...] = xc * lax.rsqrt(var + EPS) * g_ref[...] + b_ref[...]


@jax.jit
def _bert_embed(ids_flat, ttf, word_table, pos_table, type_table, gamma,
                beta):
    xg = _sc_gather(ids_flat, word_table)
    ln = pl.pallas_call(
        _ln_body,
        grid=(N_TOK // MAX_POS,),
        in_specs=[
            pl.BlockSpec((MAX_POS, HIDDEN), lambda i: (i, 0)),
            pl.BlockSpec((MAX_POS, 1), lambda i: (i, 0)),
            pl.BlockSpec((MAX_POS, HIDDEN), lambda i: (0, 0)),
            pl.BlockSpec((2, HIDDEN), lambda i: (0, 0)),
            pl.BlockSpec((1, HIDDEN), lambda i: (0, 0)),
            pl.BlockSpec((1, HIDDEN), lambda i: (0, 0)),
        ],
        out_specs=pl.BlockSpec((MAX_POS, HIDDEN), lambda i: (i, 0)),
        out_shape=jax.ShapeDtypeStruct((N_TOK, HIDDEN), jnp.float32),
        compiler_params=pltpu.CompilerParams(
            dimension_semantics=("arbitrary",)),
    )(xg, ttf, pos_table, type_table, gamma.reshape(1, HIDDEN),
      beta.reshape(1, HIDDEN))
    return ln


def kernel(input_ids, token_type_ids, word_table, pos_table, type_table,
           gamma, beta):
    B, S = input_ids.shape
    out = _bert_embed(
        input_ids.reshape(-1).astype(jnp.int32),
        token_type_ids.reshape(-1, 1).astype(jnp.float32),
        word_table, pos_table, type_table, gamma, beta)
    return out.reshape(B, S, HIDDEN)


# speedup vs baseline: 2.0582x; 1.4744x over previous
import functools

import jax
import jax.numpy as jnp
from jax import lax
from jax.experimental import pallas as pl
from jax.experimental.pallas import tpu as pltpu
from jax.experimental.pallas import tpu_sc as plsc

VOCAB = 100000
HIDDEN = 128
MAX_POS = 512
EPS = 1e-12

NC, NS, L = 2, 16, 16
NW = NC * NS
N_TOK = 64 * 512
TPW = N_TOK // NW
C = 128
NCHUNK = TPW // C


def _gather_body(ids_hbm, word_hbm, out_hbm, rows0, rows1, idx_all,
                 gsem0, gsem1, wsem0, wsem1):
    wid = lax.axis_index("s") * NC + lax.axis_index("c")
    base = wid * TPW
    pltpu.sync_copy(ids_hbm.at[pl.ds(base, TPW)], idx_all)

    def prep(c, rowsv, gsem):
        pltpu.async_copy(
            word_hbm.at[idx_all.at[pl.ds(c * C, C)]], rowsv, gsem)

    def gwait(c, rowsv, gsem):
        pltpu.make_async_copy(
            word_hbm.at[idx_all.at[pl.ds(c * C, C)]], rowsv, gsem).wait()

    def wb_start(c, rowsv, wsem):
        start = base + c * C
        pltpu.async_copy(rowsv, out_hbm.at[pl.ds(start, C)], wsem)

    def wb_wait(rowsv, wsem):
        pltpu.make_async_copy(rowsv, out_hbm.at[pl.ds(base, C)], wsem).wait()

    prep(0, rows0, gsem0)

    def pair(h, carry):
        c0 = 2 * h

        @pl.when(h > 0)
        def _():
            wb_wait(rows1, wsem1)

        prep(c0 + 1, rows1, gsem1)
        gwait(c0, rows0, gsem0)
        wb_start(c0, rows0, wsem0)

        @pl.when(h < NCHUNK // 2 - 1)
        def _():
            wb_wait(rows0, wsem0)
            prep(c0 + 2, rows0, gsem0)

        gwait(c0 + 1, rows1, gsem1)
        wb_start(c0 + 1, rows1, wsem1)
        return carry

    lax.fori_loop(0, NCHUNK // 2, pair, 0)
    wb_wait(rows0, wsem0)
    wb_wait(rows1, wsem1)


def _sc_gather(ids_flat, word_table):
    mesh = plsc.VectorSubcoreMesh(core_axis_name="c", subcore_axis_name="s")
    run = functools.partial(
        pl.kernel,
        out_type=jax.ShapeDtypeStruct((N_TOK, HIDDEN), jnp.float32),
        mesh=mesh,
        compiler_params=pltpu.CompilerParams(needs_layout_passes=False),
        scratch_types=[
            pltpu.VMEM((C, HIDDEN), jnp.float32),
            pltpu.VMEM((C, HIDDEN), jnp.float32),
            pltpu.VMEM((TPW,), jnp.int32),
            pltpu.SemaphoreType.DMA,
            pltpu.SemaphoreType.DMA,
            pltpu.SemaphoreType.DMA,
            pltpu.SemaphoreType.DMA,
        ],
    )(_gather_body)
    return run(ids_flat, word_table)


BT = 4096
NSEQ_BLK = BT // MAX_POS


def _ln_body(x_ref, ttf_ref, pos_ref, type_ref, g_ref, b_ref, o_ref):
    pos = pos_ref[...]
    t0 = type_ref[0, :][None, :]
    td = type_ref[1, :][None, :] - t0
    gv = g_ref[...]
    bv = b_ref[...]
    for s in range(NSEQ_BLK):
        sl = pl.ds(MAX_POS * s, MAX_POS)
        x = x_ref[sl, :] + pos + t0 + ttf_ref[sl, :] * td
        mu = jnp.mean(x, axis=-1, keepdims=True)
        xc = x - mu
        var = jnp.mean(xc * xc, axis=-1, keepdims=True)
        o_ref[sl, :] = xc * lax.rsqrt(var + EPS) * gv + bv


@jax.jit
def _bert_embed(ids_flat, ttf, word_table, pos_table, type_table, gamma,
                beta):
    xg = _sc_gather(ids_flat, word_table)
    ln = pl.pallas_call(
        _ln_body,
        grid=(N_TOK // BT,),
        in_specs=[
            pl.BlockSpec((BT, HIDDEN), lambda i: (i, 0)),
            pl.BlockSpec((BT, 1), lambda i: (i, 0)),
            pl.BlockSpec((MAX_POS, HIDDEN), lambda i: (0, 0)),
            pl.BlockSpec((2, HIDDEN), lambda i: (0, 0)),
            pl.BlockSpec((1, HIDDEN), lambda i: (0, 0)),
            pl.BlockSpec((1, HIDDEN), lambda i: (0, 0)),
        ],
        out_specs=pl.BlockSpec((BT, HIDDEN), lambda i: (i, 0)),
        out_shape=jax.ShapeDtypeStruct((N_TOK, HIDDEN), jnp.float32),
        compiler_params=pltpu.CompilerParams(
            dimension_semantics=("arbitrary",)),
    )(xg, ttf, pos_table, type_table, gamma.reshape(1, HIDDEN),
      beta.reshape(1, HIDDEN))
    return ln


def kernel(input_ids, token_type_ids, word_table, pos_table, type_table,
           gamma, beta):
    B, S = input_ids.shape
    out = _bert_embed(
        input_ids.reshape(-1).astype(jnp.int32),
        token_type_ids.reshape(-1, 1).astype(jnp.float32),
        word_table, pos_table, type_table, gamma, beta)
    return out.reshape(B, S, HIDDEN)


# speedup vs baseline: 2.3277x; 1.1309x over previous
import functools

import jax
import jax.numpy as jnp
from jax import lax
from jax.experimental import pallas as pl
from jax.experimental.pallas import tpu as pltpu
from jax.experimental.pallas import tpu_sc as plsc

VOCAB = 100000
HIDDEN = 128
MAX_POS = 512
EPS = 1e-12

NC, NS, L = 2, 16, 16
NW = NC * NS
N_TOK = 64 * 512
C = 128

NSPLIT = 2
N_HALF = N_TOK // NSPLIT

BT = 4096
NSEQ_BLK = BT // MAX_POS


def _make_gather_body(n_tok):
    tpw = n_tok // NW
    nchunk = tpw // C

    def body(ids_hbm, word_hbm, out_hbm, rows0, rows1, idx_all,
             gsem0, gsem1, wsem0, wsem1):
        wid = lax.axis_index("s") * NC + lax.axis_index("c")
        base = wid * tpw
        pltpu.sync_copy(ids_hbm.at[pl.ds(base, tpw)], idx_all)

        def prep(c, rowsv, gsem):
            pltpu.async_copy(
                word_hbm.at[idx_all.at[pl.ds(c * C, C)]], rowsv, gsem)

        def gwait(c, rowsv, gsem):
            pltpu.make_async_copy(
                word_hbm.at[idx_all.at[pl.ds(c * C, C)]], rowsv, gsem).wait()

        def wb_start(c, rowsv, wsem):
            start = base + c * C
            pltpu.async_copy(rowsv, out_hbm.at[pl.ds(start, C)], wsem)

        def wb_wait(rowsv, wsem):
            pltpu.make_async_copy(
                rowsv, out_hbm.at[pl.ds(base, C)], wsem).wait()

        prep(0, rows0, gsem0)

        def pair(h, carry):
            c0 = 2 * h

            @pl.when(h > 0)
            def _():
                wb_wait(rows1, wsem1)

            prep(c0 + 1, rows1, gsem1)
            gwait(c0, rows0, gsem0)
            wb_start(c0, rows0, wsem0)

            @pl.when(h < nchunk // 2 - 1)
            def _():
                wb_wait(rows0, wsem0)
                prep(c0 + 2, rows0, gsem0)

            gwait(c0 + 1, rows1, gsem1)
            wb_start(c0 + 1, rows1, wsem1)
            return carry

        lax.fori_loop(0, nchunk // 2, pair, 0)
        wb_wait(rows0, wsem0)
        wb_wait(rows1, wsem1)

    return body


def _sc_gather(ids_flat, word_table, n_tok):
    mesh = plsc.VectorSubcoreMesh(core_axis_name="c", subcore_axis_name="s")
    run = functools.partial(
        pl.kernel,
        out_type=jax.ShapeDtypeStruct((n_tok, HIDDEN), jnp.float32),
        mesh=mesh,
        compiler_params=pltpu.CompilerParams(needs_layout_passes=False),
        scratch_types=[
            pltpu.VMEM((C, HIDDEN), jnp.float32),
            pltpu.VMEM((C, HIDDEN), jnp.float32),
            pltpu.VMEM((n_tok // NW,), jnp.int32),
            pltpu.SemaphoreType.DMA,
            pltpu.SemaphoreType.DMA,
            pltpu.SemaphoreType.DMA,
            pltpu.SemaphoreType.DMA,
        ],
    )(_make_gather_body(n_tok))
    return run(ids_flat, word_table)


def _ln_math(x_ref, ttf_ref, pos_ref, type_ref, g_ref, b_ref, o_ref, obase):
    pos = pos_ref[...]
    t0 = type_ref[0, :][None, :]
    td = type_ref[1, :][None, :] - t0
    gv = g_ref[...]
    bv = b_ref[...]
    for s in range(NSEQ_BLK):
        sl = pl.ds(MAX_POS * s, MAX_POS)
        osl = pl.ds(obase + MAX_POS * s, MAX_POS)
        ttf = ttf_ref[0, :, s][:, None]
        x = x_ref[sl, :] + pos + t0 + ttf * td
        mu = jnp.mean(x, axis=-1, keepdims=True)
        xc = x - mu
        var = jnp.mean(xc * xc, axis=-1, keepdims=True)
        o_ref[osl, :] = xc * lax.rsqrt(var + EPS) * gv + bv


def _ln_body0(x_ref, ttf_ref, pos_ref, type_ref, g_ref, b_ref, o_ref):
    _ln_math(x_ref, ttf_ref, pos_ref, type_ref, g_ref, b_ref, o_ref, 0)


def _ln_body1(prev_ref, x_ref, ttf_ref, pos_ref, type_ref, g_ref, b_ref,
              o_ref):
    del prev_ref
    _ln_math(x_ref, ttf_ref, pos_ref, type_ref, g_ref, b_ref, o_ref, 0)


_LN_TAIL_SPECS = [
    pl.BlockSpec((1, MAX_POS, NSEQ_BLK), lambda i: (i, 0, 0)),
    pl.BlockSpec((MAX_POS, HIDDEN), lambda i: (0, 0)),
    pl.BlockSpec((2, HIDDEN), lambda i: (0, 0)),
    pl.BlockSpec((1, HIDDEN), lambda i: (0, 0)),
    pl.BlockSpec((1, HIDDEN), lambda i: (0, 0)),
]


@jax.jit
def _bert_embed(ids_flat, ttf3, word_table, pos_table, type_table, gamma,
                beta):
    g2 = gamma.reshape(1, HIDDEN)
    b2 = beta.reshape(1, HIDDEN)
    nblk_half = N_HALF // BT
    xg0 = _sc_gather(ids_flat[:N_HALF], word_table, N_HALF)
    xg1 = _sc_gather(ids_flat[N_HALF:], word_table, N_HALF)
    ln0 = pl.pallas_call(
        _ln_body0,
        grid=(nblk_half,),
        in_specs=[pl.BlockSpec((BT, HIDDEN), lambda i: (i, 0))]
        + _LN_TAIL_SPECS,
        out_specs=pl.BlockSpec((BT, HIDDEN), lambda i: (i, 0)),
        out_shape=jax.ShapeDtypeStruct((N_TOK, HIDDEN), jnp.float32),
        compiler_params=pltpu.CompilerParams(
            dimension_semantics=("arbitrary",)),
    )(xg0, ttf3[:nblk_half], pos_table, type_table, g2, b2)
    out = pl.pallas_call(
        _ln_body1,
        grid=(nblk_half,),
        in_specs=[pl.BlockSpec(memory_space=pltpu.MemorySpace.HBM),
                  pl.BlockSpec((BT, HIDDEN), lambda i: (i, 0))]
        + _LN_TAIL_SPECS,
        out_specs=pl.BlockSpec((BT, HIDDEN),
                               lambda i: (i + N_HALF // BT, 0)),
        out_shape=jax.ShapeDtypeStruct((N_TOK, HIDDEN), jnp.float32),
        input_output_aliases={0: 0},
        compiler_params=pltpu.CompilerParams(
            dimension_semantics=("arbitrary",)),
    )(ln0, xg1, ttf3[nblk_half:], pos_table, type_table, g2, b2)
    return out


def kernel(input_ids, token_type_ids, word_table, pos_table, type_table,
           gamma, beta):
    B, S = input_ids.shape
    out = _bert_embed(
        input_ids.reshape(-1).astype(jnp.int32),
        token_type_ids.astype(jnp.float32).reshape(
            N_TOK // BT, NSEQ_BLK, S).transpose(0, 2, 1),
        word_table, pos_table, type_table, gamma, beta)
    return out.reshape(B, S, HIDDEN)


# speedup vs baseline: 2.4409x; 1.0486x over previous
import functools

import jax
import jax.numpy as jnp
from jax import lax
from jax.experimental import pallas as pl
from jax.experimental.pallas import tpu as pltpu
from jax.experimental.pallas import tpu_sc as plsc

VOCAB = 100000
HIDDEN = 128
MAX_POS = 512
EPS = 1e-12

NC, NS, L = 2, 16, 16
NW = NC * NS
N_TOK = 64 * 512
C = 128

NSPLIT = 2
N_HALF = N_TOK // NSPLIT

BT = 4096
NSEQ_BLK = BT // MAX_POS


def _make_gather_body(n_tok):
    tpw = n_tok // NW
    nchunk = tpw // C

    def body(ids_hbm, word_hbm, out_hbm, rows0, rows1, idx_all,
             gsem0, gsem1, wsem0, wsem1):
        wid = lax.axis_index("s") * NC + lax.axis_index("c")
        base = wid * tpw
        pltpu.sync_copy(ids_hbm.at[pl.ds(base, tpw)], idx_all)

        def prep(c, rowsv, gsem):
            pltpu.async_copy(
                word_hbm.at[idx_all.at[pl.ds(c * C, C)]], rowsv, gsem)

        def gwait(c, rowsv, gsem):
            pltpu.make_async_copy(
                word_hbm.at[idx_all.at[pl.ds(c * C, C)]], rowsv, gsem).wait()

        def wb_start(c, rowsv, wsem):
            start = base + c * C
            pltpu.async_copy(rowsv, out_hbm.at[pl.ds(start, C)], wsem)

        def wb_wait(rowsv, wsem):
            pltpu.make_async_copy(
                rowsv, out_hbm.at[pl.ds(base, C)], wsem).wait()

        prep(0, rows0, gsem0)

        def pair(h, carry):
            c0 = 2 * h

            @pl.when(h > 0)
            def _():
                wb_wait(rows1, wsem1)

            prep(c0 + 1, rows1, gsem1)
            gwait(c0, rows0, gsem0)
            wb_start(c0, rows0, wsem0)

            @pl.when(h < nchunk // 2 - 1)
            def _():
                wb_wait(rows0, wsem0)
                prep(c0 + 2, rows0, gsem0)

            gwait(c0 + 1, rows1, gsem1)
            wb_start(c0 + 1, rows1, wsem1)
            return carry

        lax.fori_loop(0, nchunk // 2, pair, 0)
        wb_wait(rows0, wsem0)
        wb_wait(rows1, wsem1)

    return body


def _sc_gather(ids_flat, word_table, n_tok):
    mesh = plsc.VectorSubcoreMesh(core_axis_name="c", subcore_axis_name="s")
    run = functools.partial(
        pl.kernel,
        out_type=jax.ShapeDtypeStruct((n_tok, HIDDEN), jnp.float32),
        mesh=mesh,
        compiler_params=pltpu.CompilerParams(needs_layout_passes=False),
        scratch_types=[
            pltpu.VMEM((C, HIDDEN), jnp.float32),
            pltpu.VMEM((C, HIDDEN), jnp.float32),
            pltpu.VMEM((n_tok // NW,), jnp.int32),
            pltpu.SemaphoreType.DMA,
            pltpu.SemaphoreType.DMA,
            pltpu.SemaphoreType.DMA,
            pltpu.SemaphoreType.DMA,
        ],
    )(_make_gather_body(n_tok))
    return run(ids_flat, word_table)


def _ln_math(x_ref, ttf_ref, pos_ref, type_ref, g_ref, b_ref, o_ref, obase):
    pos = pos_ref[...]
    t0 = type_ref[0, :][None, :]
    td = type_ref[1, :][None, :] - t0
    gv = g_ref[...]
    bv = b_ref[...]
    ones = jnp.full((HIDDEN, HIDDEN), 1.0 / HIDDEN, dtype=jnp.float32)
    for s in range(NSEQ_BLK):
        sl = pl.ds(MAX_POS * s, MAX_POS)
        osl = pl.ds(obase + MAX_POS * s, MAX_POS)
        ttf = ttf_ref[0, :, s][:, None]
        x = x_ref[sl, :] + pos + t0 + ttf * td
        mu = jnp.dot(x, ones, preferred_element_type=jnp.float32)
        ex2 = jnp.dot(x * x, ones, preferred_element_type=jnp.float32)
        var = ex2 - mu * mu
        o_ref[osl, :] = (x - mu) * lax.rsqrt(var + EPS) * gv + bv


def _ln_body0(x_ref, ttf_ref, pos_ref, type_ref, g_ref, b_ref, o_ref):
    _ln_math(x_ref, ttf_ref, pos_ref, type_ref, g_ref, b_ref, o_ref, 0)


def _ln_body1(prev_ref, x_ref, ttf_ref, pos_ref, type_ref, g_ref, b_ref,
              o_ref):
    del prev_ref
    _ln_math(x_ref, ttf_ref, pos_ref, type_ref, g_ref, b_ref, o_ref, 0)


_LN_TAIL_SPECS = [
    pl.BlockSpec((1, MAX_POS, NSEQ_BLK), lambda i: (i, 0, 0)),
    pl.BlockSpec((MAX_POS, HIDDEN), lambda i: (0, 0)),
    pl.BlockSpec((2, HIDDEN), lambda i: (0, 0)),
    pl.BlockSpec((1, HIDDEN), lambda i: (0, 0)),
    pl.BlockSpec((1, HIDDEN), lambda i: (0, 0)),
]


@jax.jit
def _bert_embed(ids_flat, ttf3, word_table, pos_table, type_table, gamma,
                beta):
    g2 = gamma.reshape(1, HIDDEN)
    b2 = beta.reshape(1, HIDDEN)
    nblk_half = N_HALF // BT
    xg0 = _sc_gather(ids_flat[:N_HALF], word_table, N_HALF)
    xg1 = _sc_gather(ids_flat[N_HALF:], word_table, N_HALF)
    ln0 = pl.pallas_call(
        _ln_body0,
        grid=(nblk_half,),
        in_specs=[pl.BlockSpec((BT, HIDDEN), lambda i: (i, 0))]
        + _LN_TAIL_SPECS,
        out_specs=pl.BlockSpec((BT, HIDDEN), lambda i: (i, 0)),
        out_shape=jax.ShapeDtypeStruct((N_TOK, HIDDEN), jnp.float32),
        compiler_params=pltpu.CompilerParams(
            dimension_semantics=("arbitrary",)),
    )(xg0, ttf3[:nblk_half], pos_table, type_table, g2, b2)
    out = pl.pallas_call(
        _ln_body1,
        grid=(nblk_half,),
        in_specs=[pl.BlockSpec(memory_space=pltpu.MemorySpace.HBM),
                  pl.BlockSpec((BT, HIDDEN), lambda i: (i, 0))]
        + _LN_TAIL_SPECS,
        out_specs=pl.BlockSpec((BT, HIDDEN),
                               lambda i: (i + N_HALF // BT, 0)),
        out_shape=jax.ShapeDtypeStruct((N_TOK, HIDDEN), jnp.float32),
        input_output_aliases={0: 0},
        compiler_params=pltpu.CompilerParams(
            dimension_semantics=("arbitrary",)),
    )(ln0, xg1, ttf3[nblk_half:], pos_table, type_table, g2, b2)
    return out


def kernel(input_ids, token_type_ids, word_table, pos_table, type_table,
           gamma, beta):
    B, S = input_ids.shape
    out = _bert_embed(
        input_ids.reshape(-1).astype(jnp.int32),
        token_type_ids.astype(jnp.float32).reshape(
            N_TOK // BT, NSEQ_BLK, S).transpose(0, 2, 1),
        word_table, pos_table, type_table, gamma, beta)
    return out.reshape(B, S, HIDDEN)


# speedup vs baseline: 2.4777x; 1.0151x over previous
import functools

import jax
import jax.numpy as jnp
from jax import lax
from jax.experimental import pallas as pl
from jax.experimental.pallas import tpu as pltpu
from jax.experimental.pallas import tpu_sc as plsc

VOCAB = 100000
HIDDEN = 128
MAX_POS = 512
EPS = 1e-12

NC, NS, L = 2, 16, 16
NW = NC * NS
N_TOK = 64 * 512
C = 128

NSPLIT = 2
N_HALF = N_TOK // NSPLIT

BT = 8192
NSEQ_BLK = BT // MAX_POS


def _make_gather_body(n_tok):
    tpw = n_tok // NW
    nchunk = tpw // C

    def body(ids_hbm, word_hbm, out_hbm, rows0, rows1, idx_all,
             gsem0, gsem1, wsem0, wsem1):
        wid = lax.axis_index("s") * NC + lax.axis_index("c")
        base = wid * tpw
        pltpu.sync_copy(ids_hbm.at[pl.ds(base, tpw)], idx_all)

        def prep(c, rowsv, gsem):
            pltpu.async_copy(
                word_hbm.at[idx_all.at[pl.ds(c * C, C)]], rowsv, gsem)

        def gwait(c, rowsv, gsem):
            pltpu.make_async_copy(
                word_hbm.at[idx_all.at[pl.ds(c * C, C)]], rowsv, gsem).wait()

        def wb_start(c, rowsv, wsem):
            start = base + c * C
            pltpu.async_copy(rowsv, out_hbm.at[pl.ds(start, C)], wsem)

        def wb_wait(rowsv, wsem):
            pltpu.make_async_copy(
                rowsv, out_hbm.at[pl.ds(base, C)], wsem).wait()

        prep(0, rows0, gsem0)

        def pair(h, carry):
            c0 = 2 * h

            @pl.when(h > 0)
            def _():
                wb_wait(rows1, wsem1)

            prep(c0 + 1, rows1, gsem1)
            gwait(c0, rows0, gsem0)
            wb_start(c0, rows0, wsem0)

            @pl.when(h < nchunk // 2 - 1)
            def _():
                wb_wait(rows0, wsem0)
                prep(c0 + 2, rows0, gsem0)

            gwait(c0 + 1, rows1, gsem1)
            wb_start(c0 + 1, rows1, wsem1)
            return carry

        lax.fori_loop(0, nchunk // 2, pair, 0)
        wb_wait(rows0, wsem0)
        wb_wait(rows1, wsem1)

    return body


def _sc_gather(ids_flat, word_table, n_tok):
    mesh = plsc.VectorSubcoreMesh(core_axis_name="c", subcore_axis_name="s")
    run = functools.partial(
        pl.kernel,
        out_type=jax.ShapeDtypeStruct((n_tok, HIDDEN), jnp.float32),
        mesh=mesh,
        compiler_params=pltpu.CompilerParams(needs_layout_passes=False),
        scratch_types=[
            pltpu.VMEM((C, HIDDEN), jnp.float32),
            pltpu.VMEM((C, HIDDEN), jnp.float32),
            pltpu.VMEM((n_tok // NW,), jnp.int32),
            pltpu.SemaphoreType.DMA,
            pltpu.SemaphoreType.DMA,
            pltpu.SemaphoreType.DMA,
            pltpu.SemaphoreType.DMA,
        ],
    )(_make_gather_body(n_tok))
    return run(ids_flat, word_table)


def _ln_math(x_ref, ttf_ref, pos_ref, type_ref, g_ref, b_ref, o_ref, obase):
    pos = pos_ref[...]
    t0 = type_ref[0, :][None, :]
    td = type_ref[1, :][None, :] - t0
    gv = g_ref[...]
    bv = b_ref[...]
    ones = jnp.full((HIDDEN, HIDDEN), 1.0 / HIDDEN, dtype=jnp.float32)
    for s in range(NSEQ_BLK):
        sl = pl.ds(MAX_POS * s, MAX_POS)
        osl = pl.ds(obase + MAX_POS * s, MAX_POS)
        ttf = ttf_ref[0, :, s][:, None]
        x = x_ref[sl, :] + pos + t0 + ttf * td
        mu = jnp.dot(x, ones, preferred_element_type=jnp.float32)
        ex2 = jnp.dot(x * x, ones, preferred_element_type=jnp.float32)
        var = ex2 - mu * mu
        o_ref[osl, :] = (x - mu) * lax.rsqrt(var + EPS) * gv + bv


def _ln_body0(x_ref, ttf_ref, pos_ref, type_ref, g_ref, b_ref, o_ref):
    _ln_math(x_ref, ttf_ref, pos_ref, type_ref, g_ref, b_ref, o_ref, 0)


def _ln_body1(prev_ref, x_ref, ttf_ref, pos_ref, type_ref, g_ref, b_ref,
              o_ref):
    del prev_ref
    _ln_math(x_ref, ttf_ref, pos_ref, type_ref, g_ref, b_ref, o_ref, 0)


_LN_TAIL_SPECS = [
    pl.BlockSpec((1, MAX_POS, NSEQ_BLK), lambda i: (i, 0, 0)),
    pl.BlockSpec((MAX_POS, HIDDEN), lambda i: (0, 0)),
    pl.BlockSpec((2, HIDDEN), lambda i: (0, 0)),
    pl.BlockSpec((1, HIDDEN), lambda i: (0, 0)),
    pl.BlockSpec((1, HIDDEN), lambda i: (0, 0)),
]


@jax.jit
def _bert_embed(ids_flat, ttf3, word_table, pos_table, type_table, gamma,
                beta):
    g2 = gamma.reshape(1, HIDDEN)
    b2 = beta.reshape(1, HIDDEN)
    nblk_half = N_HALF // BT
    xg0 = _sc_gather(ids_flat[:N_HALF], word_table, N_HALF)
    xg1 = _sc_gather(ids_flat[N_HALF:], word_table, N_HALF)
    ln0 = pl.pallas_call(
        _ln_body0,
        grid=(nblk_half,),
        in_specs=[pl.BlockSpec((BT, HIDDEN), lambda i: (i, 0))]
        + _LN_TAIL_SPECS,
        out_specs=pl.BlockSpec((BT, HIDDEN), lambda i: (i, 0)),
        out_shape=jax.ShapeDtypeStruct((N_TOK, HIDDEN), jnp.float32),
        compiler_params=pltpu.CompilerParams(
            dimension_semantics=("arbitrary",)),
    )(xg0, ttf3[:nblk_half], pos_table, type_table, g2, b2)
    out = pl.pallas_call(
        _ln_body1,
        grid=(nblk_half,),
        in_specs=[pl.BlockSpec(memory_space=pltpu.MemorySpace.HBM),
                  pl.BlockSpec((BT, HIDDEN), lambda i: (i, 0))]
        + _LN_TAIL_SPECS,
        out_specs=pl.BlockSpec((BT, HIDDEN),
                               lambda i: (i + N_HALF // BT, 0)),
        out_shape=jax.ShapeDtypeStruct((N_TOK, HIDDEN), jnp.float32),
        input_output_aliases={0: 0},
        compiler_params=pltpu.CompilerParams(
            dimension_semantics=("arbitrary",)),
    )(ln0, xg1, ttf3[nblk_half:], pos_table, type_table, g2, b2)
    return out


def kernel(input_ids, token_type_ids, word_table, pos_table, type_table,
           gamma, beta):
    B, S = input_ids.shape
    out = _bert_embed(
        input_ids.reshape(-1).astype(jnp.int32),
        token_type_ids.astype(jnp.float32).reshape(
            N_TOK // BT, NSEQ_BLK, S).transpose(0, 2, 1),
        word_table, pos_table, type_table, gamma, beta)
    return out.reshape(B, S, HIDDEN)
